# baseline (speedup 1.0000x reference)
import functools
import math

import jax
import jax.numpy as jnp
from jax import lax
from jax.experimental import pallas as pl
from jax.experimental.pallas import tpu as pltpu
from jax.experimental.pallas import tpu_sc as plsc

_TOPK = 32
_NB = 1024
_NBF = float(_NB - 4)


def _fold_body(wv_ref, wo_ref, bv_ref, bo_ref, w2_ref, b2_ref):
    w2_ref[...] = jnp.dot(wv_ref[...], wo_ref[...],
                          preferred_element_type=jnp.float32)
    b2_ref[...] = jnp.dot(bv_ref[...], wo_ref[...],
                          preferred_element_type=jnp.float32) + bo_ref[...]


def _proj_body(x_ref, w_ref, b_ref, q_ref, k_ref, u_ref, *, h):
    o = (jnp.dot(x_ref[...], w_ref[...], preferred_element_type=jnp.float32)
         + b_ref[...])
    q_ref[0] = o[:, :h]
    k_ref[0] = o[:, h:2 * h]
    u_ref[0] = o[:, 2 * h:].astype(jnp.bfloat16)


def _scores_body(q_ref, k_ref, s_ref, stats_ref, *, scale):
    q = q_ref[...]
    k = k_ref[0]
    s = lax.dot_general(q, k, (((1,), (1,)), ((), ())),
                        preferred_element_type=jnp.float32) * scale
    s_ref[...] = s
    m = jnp.max(s, axis=1, keepdims=True)
    lo = jnp.min(s, axis=1, keepdims=True)
    t = s.shape[0]
    stats_ref[...] = jnp.concatenate(
        [m, lo, jnp.zeros((t, 126), jnp.float32)], axis=1)


def _sc_select_body(s_hbm, stats_hbm, out_hbm, buf, sbuf, hist, tbuf, *,
                    slen, groups):
    nc = 2
    wid = lax.axis_index("s") * nc + lax.axis_index("c")
    lanes = lax.iota(jnp.int32, 16)
    lane_base = lanes * _NB
    row_base = lanes * slen
    stat_base = lanes * 128
    zeros16 = jnp.zeros((16,), jnp.float32)
    ones16 = jnp.ones((16,), jnp.float32)
    kf = jnp.float32(float(_TOPK))

    def group_body(g, carry):
        r0 = wid * (16 * groups) + g * 16
        pltpu.sync_copy(s_hbm.at[pl.ds(r0 * slen, 16 * slen)], buf)
        pltpu.sync_copy(stats_hbm.at[pl.ds(r0 * 128, 16 * 128)], sbuf)
        m = plsc.load_gather(sbuf, [stat_base])
        lo = plsc.load_gather(sbuf, [stat_base + 1])
        k1 = _NBF / jnp.maximum(m - lo, jnp.float32(1e-30))

        def zero_hist():
            @plsc.parallel_loop(0, 2 * _NB, unroll=8)
            def _(j):
                hist[pl.ds(j * 16, 16)] = zeros16

        def run_pass1():
            @plsc.parallel_loop(0, slen, unroll=16)
            def _(e):
                bank = (e & 1) * (16 * _NB)
                v = plsc.load_gather(buf, [row_base + e])
                b1 = ((v - lo) * k1).astype(jnp.int32)
                plsc.addupdate_scatter(hist, [lane_base + bank + b1], ones16)

        def coarse_scan(kneed, acc0):
            def body(j, c):
                acc, blk, above = c
                jc = 63 - j
                cnt = zeros16
                for jf in range(16):
                    idx = lane_base + jc * 16 + jf
                    cnt = cnt + plsc.load_gather(hist, [idx])
                    cnt = cnt + plsc.load_gather(hist, [idx + 16 * _NB])
                newacc = acc + cnt
                found = jnp.logical_and(newacc >= kneed, blk < 0)
                blk = jnp.where(found, jnp.full((16,), jc, jnp.int32), blk)
                above = jnp.where(found, acc, above)
                return (newacc, blk, above)

            init = (acc0, jnp.full((16,), -1, jnp.int32), zeros16)
            _, blk, above = lax.fori_loop(0, 64, body, init)
            return blk, above

        def fine_scan(kneed, blk, above):
            def body(j, c):
                acc, fine, abv = c
                jf = 15 - j
                idx = lane_base + blk * 16 + jf
                cnt = (plsc.load_gather(hist, [idx])
                       + plsc.load_gather(hist, [idx + 16 * _NB]))
                newacc = acc + cnt
                found = jnp.logical_and(newacc >= kneed, fine < 0)
                fine = jnp.where(found, jnp.full((16,), jf, jnp.int32), fine)
                abv = jnp.where(found, acc, abv)
                return (newacc, fine, abv)

            init = (above, jnp.full((16,), -1, jnp.int32), above)
            _, fine, abv = lax.fori_loop(0, 16, body, init)
            fine = jnp.maximum(fine, jnp.zeros((16,), jnp.int32))
            return blk * 16 + fine, abv

        zero_hist()
        run_pass1()
        blk1, above1 = coarse_scan(kf, zeros16)
        beta1, above1 = fine_scan(kf, blk1, above1)
        beta1f = beta1.astype(jnp.float32)
        kneed2 = kf - above1

        zero_hist()

        @plsc.parallel_loop(0, slen, unroll=16)
        def _(e):
            bank = (e & 1) * (16 * _NB)
            v = plsc.load_gather(buf, [row_base + e])
            f = (v - lo) * k1
            b1 = f.astype(jnp.int32)
            msk = b1 == beta1
            b2 = jnp.clip((f - beta1f) * jnp.float32(_NB),
                          jnp.float32(0.0), jnp.float32(_NB - 1))
            b2i = b2.astype(jnp.int32)
            plsc.addupdate_scatter(hist, [lane_base + bank + b2i], ones16,
                                   mask=msk)
        blk2, above2 = coarse_scan(kneed2, zeros16)
        beta2, _ = fine_scan(kneed2, blk2, above2)

        tbuf[...] = beta1f * jnp.float32(_NB) + beta2.astype(jnp.float32)
        pltpu.sync_copy(tbuf, out_hbm.at[pl.ds(r0, 16)])
        return carry

    lax.fori_loop(0, groups, group_body, 0)


def _attn2_body(s_ref, t_ref, stats_ref, u_ref, b2_ref, o_ref, *, tile):
    s = s_ref[...]
    g = pl.program_id(0)
    tcol = t_ref[pl.ds(g * tile, tile), :]
    m = stats_ref[:, 0:1]
    lo = stats_ref[:, 1:2]
    k1 = _NBF / jnp.maximum(m - lo, jnp.float32(1e-30))
    f = (s - lo) * k1
    b1 = f.astype(jnp.int32)
    bi = tcol.astype(jnp.int32)
    beta1 = lax.shift_right_logical(bi, 10)
    beta2 = jnp.bitwise_and(bi, jnp.int32(_NB - 1))
    b2v = jnp.clip((f - beta1.astype(jnp.float32)) * jnp.float32(_NB),
                   jnp.float32(0.0), jnp.float32(_NB - 1)).astype(jnp.int32)
    keep = jnp.logical_or(
        b1 > beta1, jnp.logical_and(b1 == beta1, b2v >= beta2))
    p = jnp.where(keep, jnp.exp(s - m), 0.0)
    r = jnp.sum(p, axis=1, keepdims=True)
    ao = jnp.dot(p.astype(jnp.bfloat16), u_ref[0],
                 preferred_element_type=jnp.float32)
    o_ref[...] = ao / r + b2_ref[...]


def kernel(x, Wq, bq, Wk, bk, Wv, bv, Wo, bo):
    B, S, D = x.shape
    H = Wq.shape[1]
    scale = 1.0 / math.sqrt(H)
    TP = 512
    T = 512
    N = B * S

    W2, b2 = pl.pallas_call(
        _fold_body,
        out_shape=(jax.ShapeDtypeStruct((D, D), jnp.float32),
                   jax.ShapeDtypeStruct((1, D), jnp.float32)),
    )(Wv, Wo, bv.reshape(1, D), bo.reshape(1, D))

    Wcat = jnp.concatenate([Wq, Wk, W2], axis=1)
    bcat = jnp.concatenate(
        [bq, bk, jnp.zeros((D,), jnp.float32)]).reshape(1, -1)
    P = 2 * H + D
    xf = x.reshape(N, D)

    npb = S // TP
    Q, K, U = pl.pallas_call(
        functools.partial(_proj_body, h=H),
        grid=(N // TP,),
        in_specs=[
            pl.BlockSpec((TP, D), lambda i: (i, 0)),
            pl.BlockSpec((D, P), lambda i: (0, 0)),
            pl.BlockSpec((1, P), lambda i: (0, 0)),
        ],
        out_specs=(
            pl.BlockSpec((1, TP, H), lambda i: (i // npb, i % npb, 0)),
            pl.BlockSpec((1, TP, H), lambda i: (i // npb, i % npb, 0)),
            pl.BlockSpec((1, TP, D), lambda i: (i // npb, i % npb, 0)),
        ),
        out_shape=(jax.ShapeDtypeStruct((B, S, H), jnp.float32),
                   jax.ShapeDtypeStruct((B, S, H), jnp.float32),
                   jax.ShapeDtypeStruct((B, S, D), jnp.bfloat16)),
    )(xf, Wcat, bcat)

    nab = S // T
    Qf = Q.reshape(N, H)
    scores, stats = pl.pallas_call(
        functools.partial(_scores_body, scale=scale),
        grid=(N // T,),
        in_specs=[
            pl.BlockSpec((T, H), lambda g: (g, 0)),
            pl.BlockSpec((1, S, H), lambda g: (g // nab, 0, 0)),
        ],
        out_specs=(pl.BlockSpec((T, S), lambda g: (g, 0)),
                   pl.BlockSpec((T, 128), lambda g: (g, 0))),
        out_shape=(jax.ShapeDtypeStruct((N, S), jnp.float32),
                   jax.ShapeDtypeStruct((N, 128), jnp.float32)),
    )(Qf, K)

    mesh = plsc.VectorSubcoreMesh(core_axis_name="c", subcore_axis_name="s")
    thr = pl.kernel(
        functools.partial(_sc_select_body, slen=S, groups=N // (32 * 16)),
        mesh=mesh,
        compiler_params=pltpu.CompilerParams(needs_layout_passes=False),
        out_type=jax.ShapeDtypeStruct((N,), jnp.float32),
        scratch_types=[
            pltpu.VMEM((16 * S,), jnp.float32),
            pltpu.VMEM((16 * 128,), jnp.float32),
            pltpu.VMEM((2 * 16 * _NB,), jnp.float32),
            pltpu.VMEM((16,), jnp.float32),
        ],
    )(scores.reshape(N * S), stats.reshape(N * 128))

    out = pl.pallas_call(
        functools.partial(_attn2_body, tile=T),
        grid=(N // T,),
        in_specs=[
            pl.BlockSpec((T, S), lambda g: (g, 0)),
            pl.BlockSpec((N, 1), lambda g: (0, 0)),
            pl.BlockSpec((T, 128), lambda g: (g, 0)),
            pl.BlockSpec((1, S, D), lambda g: (g // nab, 0, 0)),
            pl.BlockSpec((1, D), lambda g: (0, 0)),
        ],
        out_specs=pl.BlockSpec((T, D), lambda g: (g, 0)),
        out_shape=jax.ShapeDtypeStruct((N, D), jnp.float32),
    )(scores, thr.reshape(N, 1), stats, U, b2)
    return out.reshape(B, S, D)


# speedup vs baseline: 1.1337x; 1.1337x over previous
import functools
import math

import jax
import jax.numpy as jnp
from jax import lax
from jax.experimental import pallas as pl
from jax.experimental.pallas import tpu as pltpu
from jax.experimental.pallas import tpu_sc as plsc

_TOPK = 32
_NB = 1024
_NBF = float(_NB - 4)


def _fold_body(wv_ref, wo_ref, bv_ref, bo_ref, w2_ref, b2_ref):
    w2_ref[...] = jnp.dot(wv_ref[...], wo_ref[...],
                          preferred_element_type=jnp.float32)
    b2_ref[...] = jnp.dot(bv_ref[...], wo_ref[...],
                          preferred_element_type=jnp.float32) + bo_ref[...]


def _proj_body(x_ref, w_ref, b_ref, q_ref, k_ref, u_ref, *, h):
    o = (jnp.dot(x_ref[...], w_ref[...], preferred_element_type=jnp.float32)
         + b_ref[...])
    q_ref[0] = o[:, :h]
    k_ref[0] = o[:, h:2 * h]
    u_ref[0] = o[:, 2 * h:].astype(jnp.bfloat16)


def _scores_body(q_ref, k_ref, s_ref, stats_ref, *, scale):
    q = q_ref[...]
    k = k_ref[0]
    s = lax.dot_general(q, k, (((1,), (1,)), ((), ())),
                        preferred_element_type=jnp.float32) * scale
    s_ref[...] = s
    m = jnp.max(s, axis=1, keepdims=True)
    lo = jnp.min(s, axis=1, keepdims=True)
    t = s.shape[0]
    stats_ref[...] = jnp.concatenate(
        [m, lo, jnp.zeros((t, 126), jnp.float32)], axis=1)


def _sc_select_body(s_hbm, stats_hbm, out_hbm, buf, sbuf, hist, tbuf, *,
                    slen, groups):
    nc = 2
    wid = lax.axis_index("s") * nc + lax.axis_index("c")
    lanes = lax.iota(jnp.int32, 16)
    lane_base = lanes * _NB
    row_base = lanes * slen
    stat_base = lanes * 128
    zeros16 = jnp.zeros((16,), jnp.float32)
    ones16 = jnp.ones((16,), jnp.float32)
    kf = jnp.float32(float(_TOPK))

    def group_body(g, carry):
        r0 = wid * (16 * groups) + g * 16
        pltpu.sync_copy(s_hbm.at[pl.ds(r0 * slen, 16 * slen)], buf)
        pltpu.sync_copy(stats_hbm.at[pl.ds(r0 * 128, 16 * 128)], sbuf)
        m = plsc.load_gather(sbuf, [stat_base])
        lo = plsc.load_gather(sbuf, [stat_base + 1])
        k1 = _NBF / jnp.maximum(m - lo, jnp.float32(1e-30))

        def zero_hist():
            @plsc.parallel_loop(0, _NB, unroll=8)
            def _(j):
                hist[pl.ds(j * 16, 16)] = zeros16

        def run_pass1():
            @plsc.parallel_loop(0, slen, unroll=16)
            def _(e):
                v = plsc.load_gather(buf, [row_base + e])
                b1 = ((v - lo) * k1).astype(jnp.int32)
                plsc.addupdate_scatter(hist, [lane_base + b1], ones16)

        def coarse_scan(kneed, acc0):
            def body(j, c):
                acc, blk, above = c
                jc = 63 - j
                cnt = zeros16
                for jf in range(16):
                    idx = lane_base + jc * 16 + jf
                    cnt = cnt + plsc.load_gather(hist, [idx])
                newacc = acc + cnt
                found = jnp.logical_and(newacc >= kneed, blk < 0)
                blk = jnp.where(found, jnp.full((16,), jc, jnp.int32), blk)
                above = jnp.where(found, acc, above)
                return (newacc, blk, above)

            init = (acc0, jnp.full((16,), -1, jnp.int32), zeros16)
            _, blk, above = lax.fori_loop(0, 64, body, init)
            return blk, above

        def fine_scan(kneed, blk, above):
            def body(j, c):
                acc, fine, abv = c
                jf = 15 - j
                idx = lane_base + blk * 16 + jf
                cnt = plsc.load_gather(hist, [idx])
                newacc = acc + cnt
                found = jnp.logical_and(newacc >= kneed, fine < 0)
                fine = jnp.where(found, jnp.full((16,), jf, jnp.int32), fine)
                abv = jnp.where(found, acc, abv)
                return (newacc, fine, abv)

            init = (above, jnp.full((16,), -1, jnp.int32), above)
            _, fine, abv = lax.fori_loop(0, 16, body, init)
            fine = jnp.maximum(fine, jnp.zeros((16,), jnp.int32))
            return blk * 16 + fine, abv

        zero_hist()
        run_pass1()
        blk1, above1 = coarse_scan(kf, zeros16)
        beta1, above1 = fine_scan(kf, blk1, above1)
        beta1f = beta1.astype(jnp.float32)
        kneed2 = kf - above1

        zero_hist()

        @plsc.parallel_loop(0, slen, unroll=16)
        def _(e):
            v = plsc.load_gather(buf, [row_base + e])
            f = (v - lo) * k1
            b1 = f.astype(jnp.int32)
            msk = b1 == beta1
            b2 = jnp.clip((f - beta1f) * jnp.float32(_NB),
                          jnp.float32(0.0), jnp.float32(_NB - 1))
            b2i = b2.astype(jnp.int32)
            plsc.addupdate_scatter(hist, [lane_base + b2i], ones16,
                                   mask=msk)
        blk2, above2 = coarse_scan(kneed2, zeros16)
        beta2, _ = fine_scan(kneed2, blk2, above2)

        tbuf[...] = beta1f * jnp.float32(_NB) + beta2.astype(jnp.float32)
        pltpu.sync_copy(tbuf, out_hbm.at[pl.ds(r0, 16)])
        return carry

    lax.fori_loop(0, groups, group_body, 0)


def _attn2_body(s_ref, t_ref, stats_ref, u_ref, b2_ref, o_ref, *, tile):
    s = s_ref[...]
    g = pl.program_id(0)
    tcol = t_ref[pl.ds(g * tile, tile), :]
    m = stats_ref[:, 0:1]
    lo = stats_ref[:, 1:2]
    k1 = _NBF / jnp.maximum(m - lo, jnp.float32(1e-30))
    f = (s - lo) * k1
    b1 = f.astype(jnp.int32)
    bi = tcol.astype(jnp.int32)
    beta1 = lax.shift_right_logical(bi, 10)
    beta2 = jnp.bitwise_and(bi, jnp.int32(_NB - 1))
    b2v = jnp.clip((f - beta1.astype(jnp.float32)) * jnp.float32(_NB),
                   jnp.float32(0.0), jnp.float32(_NB - 1)).astype(jnp.int32)
    keep = jnp.logical_or(
        b1 > beta1, jnp.logical_and(b1 == beta1, b2v >= beta2))
    p = jnp.where(keep, jnp.exp(s - m), 0.0)
    r = jnp.sum(p, axis=1, keepdims=True)
    ao = jnp.dot(p.astype(jnp.bfloat16), u_ref[0],
                 preferred_element_type=jnp.float32)
    o_ref[...] = ao / r + b2_ref[...]


def kernel(x, Wq, bq, Wk, bk, Wv, bv, Wo, bo):
    B, S, D = x.shape
    H = Wq.shape[1]
    scale = 1.0 / math.sqrt(H)
    TP = 512
    T = 512
    N = B * S

    W2, b2 = pl.pallas_call(
        _fold_body,
        out_shape=(jax.ShapeDtypeStruct((D, D), jnp.float32),
                   jax.ShapeDtypeStruct((1, D), jnp.float32)),
    )(Wv, Wo, bv.reshape(1, D), bo.reshape(1, D))

    Wcat = jnp.concatenate([Wq, Wk, W2], axis=1)
    bcat = jnp.concatenate(
        [bq, bk, jnp.zeros((D,), jnp.float32)]).reshape(1, -1)
    P = 2 * H + D
    xf = x.reshape(N, D)

    npb = S // TP
    Q, K, U = pl.pallas_call(
        functools.partial(_proj_body, h=H),
        grid=(N // TP,),
        in_specs=[
            pl.BlockSpec((TP, D), lambda i: (i, 0)),
            pl.BlockSpec((D, P), lambda i: (0, 0)),
            pl.BlockSpec((1, P), lambda i: (0, 0)),
        ],
        out_specs=(
            pl.BlockSpec((1, TP, H), lambda i: (i // npb, i % npb, 0)),
            pl.BlockSpec((1, TP, H), lambda i: (i // npb, i % npb, 0)),
            pl.BlockSpec((1, TP, D), lambda i: (i // npb, i % npb, 0)),
        ),
        out_shape=(jax.ShapeDtypeStruct((B, S, H), jnp.float32),
                   jax.ShapeDtypeStruct((B, S, H), jnp.float32),
                   jax.ShapeDtypeStruct((B, S, D), jnp.bfloat16)),
    )(xf, Wcat, bcat)

    nab = S // T
    Qf = Q.reshape(N, H)
    scores, stats = pl.pallas_call(
        functools.partial(_scores_body, scale=scale),
        grid=(N // T,),
        in_specs=[
            pl.BlockSpec((T, H), lambda g: (g, 0)),
            pl.BlockSpec((1, S, H), lambda g: (g // nab, 0, 0)),
        ],
        out_specs=(pl.BlockSpec((T, S), lambda g: (g, 0)),
                   pl.BlockSpec((T, 128), lambda g: (g, 0))),
        out_shape=(jax.ShapeDtypeStruct((N, S), jnp.float32),
                   jax.ShapeDtypeStruct((N, 128), jnp.float32)),
    )(Qf, K)

    mesh = plsc.VectorSubcoreMesh(core_axis_name="c", subcore_axis_name="s")
    thr = pl.kernel(
        functools.partial(_sc_select_body, slen=S, groups=N // (32 * 16)),
        mesh=mesh,
        compiler_params=pltpu.CompilerParams(needs_layout_passes=False),
        out_type=jax.ShapeDtypeStruct((N,), jnp.float32),
        scratch_types=[
            pltpu.VMEM((16 * S,), jnp.float32),
            pltpu.VMEM((16 * 128,), jnp.float32),
            pltpu.VMEM((16 * _NB,), jnp.float32),
            pltpu.VMEM((16,), jnp.float32),
        ],
    )(scores.reshape(N * S), stats.reshape(N * 128))

    out = pl.pallas_call(
        functools.partial(_attn2_body, tile=T),
        grid=(N // T,),
        in_specs=[
            pl.BlockSpec((T, S), lambda g: (g, 0)),
            pl.BlockSpec((N, 1), lambda g: (0, 0)),
            pl.BlockSpec((T, 128), lambda g: (g, 0)),
            pl.BlockSpec((1, S, D), lambda g: (g // nab, 0, 0)),
            pl.BlockSpec((1, D), lambda g: (0, 0)),
        ],
        out_specs=pl.BlockSpec((T, D), lambda g: (g, 0)),
        out_shape=jax.ShapeDtypeStruct((N, D), jnp.float32),
    )(scores, thr.reshape(N, 1), stats, U, b2)
    return out.reshape(B, S, D)
